# baseline (speedup 1.0000x reference)
import functools

import jax
import jax.numpy as jnp
from jax import lax
from jax.experimental import pallas as pl
from jax.experimental.pallas import tpu as pltpu
from jax.experimental.pallas import tpu_sc as plsc

VOCAB_SIZE = 1000000
EMB_DIM = 32
BATCH_SIZE = 16384


def _build_sc_gather():
    info = plsc.get_sparse_core_info()
    num_cores, num_subcores = info.num_cores, info.num_subcores
    num_workers = num_cores * num_subcores
    b_per_w = BATCH_SIZE // num_workers
    mesh = plsc.VectorSubcoreMesh(core_axis_name="c", subcore_axis_name="s")

    @functools.partial(
        pl.kernel,
        mesh=mesh,
        compiler_params=pltpu.CompilerParams(use_tc_tiling_on_sc=False),
        out_type=[
            jax.ShapeDtypeStruct((EMB_DIM, BATCH_SIZE), jnp.float32),
            jax.ShapeDtypeStruct((EMB_DIM, BATCH_SIZE), jnp.float32),
        ],
        scratch_types=[
            pltpu.VMEM((b_per_w,), jnp.int32),
            pltpu.VMEM((b_per_w,), jnp.int32),
            pltpu.VMEM((EMB_DIM, b_per_w), jnp.float32),
            pltpu.VMEM((EMB_DIM, b_per_w), jnp.float32),
            pltpu.SemaphoreType.DMA,
            pltpu.SemaphoreType.DMA,
        ],
    )
    def sc_gather(targets_hbm, contexts_hbm, ttab_hbm, ctab_hbm,
                  tout_hbm, cout_hbm,
                  tidx_v, cidx_v, tvout_v, cvout_v, sem_t, sem_c):
        wid = lax.axis_index("s") * num_cores + lax.axis_index("c")
        base = wid * b_per_w
        pltpu.sync_copy(targets_hbm.at[pl.ds(base, b_per_w)], tidx_v)
        pltpu.sync_copy(contexts_hbm.at[pl.ds(base, b_per_w)], cidx_v)

        def body(g, carry):
            s = pl.ds(g * 16, 16)
            idx_t = tidx_v[s]
            idx_c = cidx_v[s]
            for j in range(EMB_DIM):
                pltpu.async_copy(
                    ttab_hbm.at[j].at[idx_t], tvout_v.at[j, s], sem_t)
                pltpu.async_copy(
                    ctab_hbm.at[j].at[idx_c], cvout_v.at[j, s], sem_c)
            return carry
        lax.fori_loop(0, b_per_w // 16, body, 0)

        pltpu.make_async_copy(
            tout_hbm.at[:, pl.ds(base, b_per_w)], tvout_v, sem_t).wait()
        pltpu.make_async_copy(
            cout_hbm.at[:, pl.ds(base, b_per_w)], cvout_v, sem_c).wait()

        pltpu.sync_copy(tvout_v, tout_hbm.at[:, pl.ds(base, b_per_w)])
        pltpu.sync_copy(cvout_v, cout_hbm.at[:, pl.ds(base, b_per_w)])

    return sc_gather


_sc_gather = _build_sc_gather()


@jax.jit
def kernel(targets, contexts, target_table, context_table):
    t_emb_t, c_emb_t = _sc_gather(
        targets.astype(jnp.int32), contexts.astype(jnp.int32),
        target_table.T, context_table.T)
    return (t_emb_t.T, c_emb_t.T)


# speedup vs baseline: 5.5933x; 5.5933x over previous
import functools

import jax
import jax.numpy as jnp
from jax import lax
from jax.experimental import pallas as pl
from jax.experimental.pallas import tpu as pltpu
from jax.experimental.pallas import tpu_sc as plsc

VOCAB_SIZE = 1000000
EMB_DIM = 32
BATCH_SIZE = 16384


def _build_sc_gather():
    info = plsc.get_sparse_core_info()
    num_cores, num_subcores = info.num_cores, info.num_subcores
    num_workers = num_cores * num_subcores
    b_per_w = BATCH_SIZE // num_workers
    mesh = plsc.VectorSubcoreMesh(core_axis_name="c", subcore_axis_name="s")

    @functools.partial(
        pl.kernel,
        mesh=mesh,
        compiler_params=pltpu.CompilerParams(use_tc_tiling_on_sc=False),
        out_type=[
            jax.ShapeDtypeStruct((BATCH_SIZE, EMB_DIM), jnp.float32),
            jax.ShapeDtypeStruct((BATCH_SIZE, EMB_DIM), jnp.float32),
        ],
        scratch_types=[
            pltpu.VMEM((b_per_w,), jnp.int32),
            pltpu.VMEM((b_per_w,), jnp.int32),
            pltpu.VMEM((b_per_w, EMB_DIM), jnp.float32),
            pltpu.VMEM((b_per_w, EMB_DIM), jnp.float32),
            pltpu.SemaphoreType.DMA,
            pltpu.SemaphoreType.DMA,
        ],
    )
    def sc_gather(targets_hbm, contexts_hbm, ttable_hbm, ctable_hbm,
                  tout_hbm, cout_hbm,
                  tidx_v, cidx_v, trows_v, crows_v, sem_t, sem_c):
        wid = lax.axis_index("s") * num_cores + lax.axis_index("c")
        base = wid * b_per_w
        pltpu.sync_copy(targets_hbm.at[pl.ds(base, b_per_w)], tidx_v)
        pltpu.sync_copy(contexts_hbm.at[pl.ds(base, b_per_w)], cidx_v)
        cp_t = pltpu.async_copy(ttable_hbm.at[tidx_v], trows_v, sem_t)
        cp_c = pltpu.async_copy(ctable_hbm.at[cidx_v], crows_v, sem_c)
        cp_t.wait()
        pltpu.sync_copy(trows_v, tout_hbm.at[pl.ds(base, b_per_w)])
        cp_c.wait()
        pltpu.sync_copy(crows_v, cout_hbm.at[pl.ds(base, b_per_w)])

    return sc_gather


_sc_gather = _build_sc_gather()


@jax.jit
def kernel(targets, contexts, target_table, context_table):
    target_emb, context_emb = _sc_gather(
        targets.astype(jnp.int32), contexts.astype(jnp.int32),
        target_table, context_table)
    return (target_emb, context_emb)
